# Initial kernel scaffold; baseline (speedup 1.0000x reference)
import jax
import jax.numpy as jnp
from jax.experimental import pallas as pl


def kernel(x, edge_index, batch, enc_W, enc_b, conv_W, conv_b, dec_W0, dec_b0, dec_W1, dec_b1):
    raise NotImplementedError("write your pallas kernel here")


# speedup vs baseline: 12.4604x; 12.4604x over previous
import functools

import jax
import jax.numpy as jnp
from jax import lax
from jax.experimental import pallas as pl
from jax.experimental.pallas import tpu as pltpu
from jax.experimental.pallas import tpu_sc as plsc

N = 10000
E = 320000
DH = 128
DOUT = 64
G = 16

NC = 2
NS = 16
NW = NC * NS
EPW = E // NW
C = 80
NCHUNK = EPW // C
NP = 10240
ROWS = NP // NS

RB = 2000
NBLK = N // RB


@functools.cache
def _sc_kernels():
    mesh = plsc.VectorSubcoreMesh(
        core_axis_name="c", subcore_axis_name="s", num_cores=NC, num_subcores=NS
    )

    @functools.partial(
        pl.kernel,
        out_type=jax.ShapeDtypeStruct((NC, NP, DH), jnp.float32),
        mesh=mesh,
        scratch_types=[
            pltpu.VMEM((NCHUNK, C), jnp.int32),
            pltpu.VMEM((C, DH), jnp.float32),
            pltpu.VMEM_SHARED((NP, DH), jnp.float32),
        ],
    )
    def _sc_deg(dst_hbm, ones_hbm, zeros_hbm, out_hbm, idx_v, ones_v, deg_s):
        c = lax.axis_index("c")
        s = lax.axis_index("s")
        wid = s * NC + c
        pltpu.sync_copy(zeros_hbm.at[pl.ds(s * ROWS, ROWS)],
                        deg_s.at[pl.ds(s * ROWS, ROWS)])
        pltpu.sync_copy(ones_hbm, ones_v)
        pltpu.sync_copy(dst_hbm.at[wid], idx_v)
        plsc.subcore_barrier()

        @pl.loop(0, NCHUNK)
        def _(j):
            pltpu.sync_copy(ones_v, deg_s.at[idx_v.at[j]], add=True)

        plsc.subcore_barrier()
        pltpu.sync_copy(deg_s.at[pl.ds(s * ROWS, ROWS)],
                        out_hbm.at[c, pl.ds(s * ROWS, ROWS)])

    @functools.partial(
        pl.kernel,
        out_type=jax.ShapeDtypeStruct((NC, NP, DH), jnp.float32),
        mesh=mesh,
        scratch_types=[
            pltpu.VMEM((NCHUNK, C), jnp.int32),
            pltpu.VMEM((NCHUNK, C), jnp.int32),
            pltpu.VMEM((C, DH), jnp.float32),
            pltpu.VMEM_SHARED((NP, DH), jnp.float32),
            pltpu.SemaphoreType.DMA,
        ],
    )
    def _sc_agg(u_hbm, src_hbm, dst_hbm, zeros_hbm, out_hbm,
                src_v, dst_v, buf0, agg_s, sem0):
        c = lax.axis_index("c")
        s = lax.axis_index("s")
        wid = s * NC + c
        pltpu.sync_copy(zeros_hbm.at[pl.ds(s * ROWS, ROWS)],
                        agg_s.at[pl.ds(s * ROWS, ROWS)])
        pltpu.sync_copy(src_hbm.at[wid], src_v)
        pltpu.sync_copy(dst_hbm.at[wid], dst_v)
        plsc.subcore_barrier()

        @pl.loop(0, NCHUNK)
        def _(j):
            pltpu.async_copy(u_hbm.at[src_v.at[j]], buf0, sem0).wait()
            pltpu.sync_copy(buf0, agg_s.at[dst_v.at[j]], add=True)

        plsc.subcore_barrier()
        pltpu.sync_copy(agg_s.at[pl.ds(s * ROWS, ROWS)],
                        out_hbm.at[c, pl.ds(s * ROWS, ROWS)])

    return _sc_deg, _sc_agg


def _tc_enc_body(x_ref, w0_ref, b0_ref, w1_ref, b1_ref, wc_ref,
                 deg0_ref, deg1_ref, u_ref, dinv_ref):
    h = jnp.dot(x_ref[...], w0_ref[...], preferred_element_type=jnp.float32)
    h = jnp.maximum(h + b0_ref[...], 0.0)
    h = jnp.dot(h, w1_ref[...], preferred_element_type=jnp.float32) + b1_ref[...]
    dinv = lax.rsqrt(deg0_ref[...][:, :1] + deg1_ref[...][:, :1] + 1.0)
    t = jnp.dot(h, wc_ref[...], preferred_element_type=jnp.float32)
    u_ref[...] = t * dinv
    dinv_ref[...] = jnp.broadcast_to(dinv, (RB, 16))


_tc_enc = pl.pallas_call(
    _tc_enc_body,
    grid=(NBLK,),
    in_specs=[
        pl.BlockSpec((RB, DH), lambda i: (i, 0)),
        pl.BlockSpec((DH, DH), lambda i: (0, 0)),
        pl.BlockSpec((1, DH), lambda i: (0, 0)),
        pl.BlockSpec((DH, DH), lambda i: (0, 0)),
        pl.BlockSpec((1, DH), lambda i: (0, 0)),
        pl.BlockSpec((DH, DH), lambda i: (0, 0)),
        pl.BlockSpec((RB, DH), lambda i: (i, 0)),
        pl.BlockSpec((RB, DH), lambda i: (i, 0)),
    ],
    out_specs=[
        pl.BlockSpec((RB, DH), lambda i: (i, 0)),
        pl.BlockSpec((RB, 16), lambda i: (i, 0)),
    ],
    out_shape=[
        jax.ShapeDtypeStruct((N, DH), jnp.float32),
        jax.ShapeDtypeStruct((N, 16), jnp.float32),
    ],
)


def _tc_mid_body(a0_ref, a1_ref, u_ref, dinv_ref, b_ref, w_ref, uo_ref):
    dinv = dinv_ref[...][:, :1]
    h = (a0_ref[...] + a1_ref[...] + u_ref[...]) * dinv + b_ref[...]
    h = jnp.maximum(h, 0.0)
    t = jnp.dot(h, w_ref[...], preferred_element_type=jnp.float32)
    uo_ref[...] = t * dinv


_tc_mid = pl.pallas_call(
    _tc_mid_body,
    grid=(NBLK,),
    in_specs=[
        pl.BlockSpec((RB, DH), lambda i: (i, 0)),
        pl.BlockSpec((RB, DH), lambda i: (i, 0)),
        pl.BlockSpec((RB, DH), lambda i: (i, 0)),
        pl.BlockSpec((RB, 16), lambda i: (i, 0)),
        pl.BlockSpec((1, DH), lambda i: (0, 0)),
        pl.BlockSpec((DH, DH), lambda i: (0, 0)),
    ],
    out_specs=pl.BlockSpec((RB, DH), lambda i: (i, 0)),
    out_shape=jax.ShapeDtypeStruct((N, DH), jnp.float32),
)


def _tc_fin_body(a0_ref, a1_ref, u_ref, dinv_ref, b_ref, batch_ref,
                 dw0_ref, db0_ref, dw1_ref, db1_ref, out_ref, pool_ref):
    i = pl.program_id(0)

    @pl.when(i == 0)
    def _():
        pool_ref[...] = jnp.zeros_like(pool_ref)

    dinv = dinv_ref[...][:, :1]
    h = (a0_ref[...] + a1_ref[...] + u_ref[...]) * dinv + b_ref[...]
    h = jnp.maximum(h, 0.0)
    row = batch_ref[0, 0, :]
    seg = (row[None, :] == lax.broadcasted_iota(jnp.int32, (G, RB), 0))
    seg = seg.astype(jnp.float32)
    pool_ref[...] += jnp.dot(seg, h, preferred_element_type=jnp.float32)

    @pl.when(i == NBLK - 1)
    def _():
        z = jnp.dot(pool_ref[...], dw0_ref[...],
                    preferred_element_type=jnp.float32) + db0_ref[...]
        z = jnp.maximum(z, 0.0)
        out_ref[...] = jnp.dot(z, dw1_ref[...],
                               preferred_element_type=jnp.float32) + db1_ref[...]


_tc_fin = pl.pallas_call(
    _tc_fin_body,
    grid=(NBLK,),
    in_specs=[
        pl.BlockSpec((RB, DH), lambda i: (i, 0)),
        pl.BlockSpec((RB, DH), lambda i: (i, 0)),
        pl.BlockSpec((RB, DH), lambda i: (i, 0)),
        pl.BlockSpec((RB, 16), lambda i: (i, 0)),
        pl.BlockSpec((1, DH), lambda i: (0, 0)),
        pl.BlockSpec((1, 1, RB), lambda i: (i, 0, 0)),
        pl.BlockSpec((DH, DH), lambda i: (0, 0)),
        pl.BlockSpec((1, DH), lambda i: (0, 0)),
        pl.BlockSpec((DH, DOUT), lambda i: (0, 0)),
        pl.BlockSpec((1, DOUT), lambda i: (0, 0)),
    ],
    out_specs=pl.BlockSpec((G, DOUT), lambda i: (0, 0)),
    out_shape=jax.ShapeDtypeStruct((G, DOUT), jnp.float32),
    scratch_shapes=[pltpu.VMEM((G, DH), jnp.float32)],
)


def kernel(x, edge_index, batch, enc_W, enc_b, conv_W, conv_b,
           dec_W0, dec_b0, dec_W1, dec_b1):
    src = edge_index[0].reshape(NW, NCHUNK, C)
    dst = edge_index[1].reshape(NW, NCHUNK, C)
    zeros128 = jnp.zeros((NP, DH), jnp.float32)
    ones128 = jnp.ones((C, DH), jnp.float32)
    batch3 = batch.reshape(NBLK, 1, RB)

    _sc_deg, _sc_agg = _sc_kernels()
    deg = _sc_deg(dst, ones128, zeros128)
    u, dinv = _tc_enc(x, enc_W[0], enc_b[0:1], enc_W[1], enc_b[1:2],
                      conv_W[0], deg[0], deg[1])

    for i in range(3):
        agg = _sc_agg(u, src, dst, zeros128)
        if i < 2:
            u = _tc_mid(agg[0], agg[1], u, dinv, conv_b[i:i + 1],
                        conv_W[i + 1])
        else:
            out = _tc_fin(agg[0], agg[1], u, dinv, conv_b[i:i + 1], batch3,
                          dec_W0, dec_b0[None], dec_W1, dec_b1[None])
    return out


# speedup vs baseline: 17.7202x; 1.4221x over previous
import functools

import jax
import jax.numpy as jnp
from jax import lax
from jax.experimental import pallas as pl
from jax.experimental.pallas import tpu as pltpu
from jax.experimental.pallas import tpu_sc as plsc

N = 10000
E = 320000
DH = 128
DOUT = 64
G = 16

NC = 2
NS = 16
NW = NC * NS
EPW = E // NW
C = 80
NCHUNK = EPW // C
SUP = 25
NSUP = NCHUNK // SUP
NP = 10240
ROWS = NP // NS

RB = 2000
NBLK = N // RB


@functools.cache
def _sc_kernels():
    mesh = plsc.VectorSubcoreMesh(
        core_axis_name="c", subcore_axis_name="s", num_cores=NC, num_subcores=NS
    )

    @functools.partial(
        pl.kernel,
        out_type=jax.ShapeDtypeStruct((NC, NP, DH), jnp.float32),
        mesh=mesh,
        scratch_types=[
            pltpu.VMEM((NCHUNK, C), jnp.int32),
            pltpu.VMEM((C, DH), jnp.float32),
            pltpu.VMEM_SHARED((NP, DH), jnp.float32),
        ],
    )
    def _sc_deg(dst_hbm, ones_hbm, zeros_hbm, out_hbm, idx_v, ones_v, deg_s):
        c = lax.axis_index("c")
        s = lax.axis_index("s")
        wid = s * NC + c
        pltpu.sync_copy(zeros_hbm.at[pl.ds(s * ROWS, ROWS)],
                        deg_s.at[pl.ds(s * ROWS, ROWS)])
        pltpu.sync_copy(ones_hbm, ones_v)
        pltpu.sync_copy(dst_hbm.at[wid], idx_v)
        plsc.subcore_barrier()

        @pl.loop(0, NCHUNK)
        def _(j):
            pltpu.sync_copy(ones_v, deg_s.at[idx_v.at[j]], add=True)

        plsc.subcore_barrier()
        pltpu.sync_copy(deg_s.at[pl.ds(s * ROWS, ROWS)],
                        out_hbm.at[c, pl.ds(s * ROWS, ROWS)])

    @functools.partial(
        pl.kernel,
        out_type=jax.ShapeDtypeStruct((NC, NP, DH), jnp.float32),
        mesh=mesh,
        scratch_types=[
            pltpu.VMEM((SUP, C), jnp.int32),
            pltpu.VMEM((SUP, C), jnp.int32),
            pltpu.VMEM((C, DH), jnp.float32),
            pltpu.VMEM((C, DH), jnp.float32),
            pltpu.VMEM_SHARED((NP, DH), jnp.float32),
            pltpu.SemaphoreType.DMA,
            pltpu.SemaphoreType.DMA,
        ],
    )
    def _sc_agg(u_hbm, src_hbm, dst_hbm, zeros_hbm, out_hbm,
                src_v, dst_v, buf0, buf1, agg_s, sem0, sem1):
        c = lax.axis_index("c")
        s = lax.axis_index("s")
        wid = s * NC + c
        pltpu.sync_copy(zeros_hbm.at[pl.ds(s * ROWS, ROWS)],
                        agg_s.at[pl.ds(s * ROWS, ROWS)])
        plsc.subcore_barrier()

        @pl.loop(0, NSUP)
        def _(g):
            pltpu.sync_copy(src_hbm.at[wid, g], src_v)
            pltpu.sync_copy(dst_hbm.at[wid, g], dst_v)
            pltpu.async_copy(u_hbm.at[src_v.at[0]], buf0, sem0)

            @pl.loop(0, (SUP - 1) // 2)
            def _(jh):
                j = 2 * jh
                pltpu.async_copy(u_hbm.at[src_v.at[j + 1]], buf1, sem1)
                pltpu.make_async_copy(u_hbm.at[src_v.at[j]], buf0, sem0).wait()
                pltpu.sync_copy(buf0, agg_s.at[dst_v.at[j]], add=True)
                pltpu.async_copy(u_hbm.at[src_v.at[j + 2]], buf0, sem0)
                pltpu.make_async_copy(u_hbm.at[src_v.at[j + 1]], buf1, sem1).wait()
                pltpu.sync_copy(buf1, agg_s.at[dst_v.at[j + 1]], add=True)

            pltpu.make_async_copy(u_hbm.at[src_v.at[SUP - 1]], buf0, sem0).wait()
            pltpu.sync_copy(buf0, agg_s.at[dst_v.at[SUP - 1]], add=True)

        plsc.subcore_barrier()
        pltpu.sync_copy(agg_s.at[pl.ds(s * ROWS, ROWS)],
                        out_hbm.at[c, pl.ds(s * ROWS, ROWS)])

    return _sc_deg, _sc_agg


def _tc_enc_body(x_ref, w0_ref, b0_ref, w1_ref, b1_ref, wc_ref,
                 deg0_ref, deg1_ref, u_ref, dinv_ref):
    h = jnp.dot(x_ref[...], w0_ref[...], preferred_element_type=jnp.float32)
    h = jnp.maximum(h + b0_ref[...], 0.0)
    h = jnp.dot(h, w1_ref[...], preferred_element_type=jnp.float32) + b1_ref[...]
    dinv = lax.rsqrt(deg0_ref[...][:, :1] + deg1_ref[...][:, :1] + 1.0)
    t = jnp.dot(h, wc_ref[...], preferred_element_type=jnp.float32)
    u_ref[...] = t * dinv
    dinv_ref[...] = jnp.broadcast_to(dinv, (RB, 16))


_tc_enc = pl.pallas_call(
    _tc_enc_body,
    grid=(NBLK,),
    in_specs=[
        pl.BlockSpec((RB, DH), lambda i: (i, 0)),
        pl.BlockSpec((DH, DH), lambda i: (0, 0)),
        pl.BlockSpec((1, DH), lambda i: (0, 0)),
        pl.BlockSpec((DH, DH), lambda i: (0, 0)),
        pl.BlockSpec((1, DH), lambda i: (0, 0)),
        pl.BlockSpec((DH, DH), lambda i: (0, 0)),
        pl.BlockSpec((RB, DH), lambda i: (i, 0)),
        pl.BlockSpec((RB, DH), lambda i: (i, 0)),
    ],
    out_specs=[
        pl.BlockSpec((RB, DH), lambda i: (i, 0)),
        pl.BlockSpec((RB, 16), lambda i: (i, 0)),
    ],
    out_shape=[
        jax.ShapeDtypeStruct((N, DH), jnp.float32),
        jax.ShapeDtypeStruct((N, 16), jnp.float32),
    ],
)


def _tc_mid_body(a0_ref, a1_ref, u_ref, dinv_ref, b_ref, w_ref, uo_ref):
    dinv = dinv_ref[...][:, :1]
    h = (a0_ref[...] + a1_ref[...] + u_ref[...]) * dinv + b_ref[...]
    h = jnp.maximum(h, 0.0)
    t = jnp.dot(h, w_ref[...], preferred_element_type=jnp.float32)
    uo_ref[...] = t * dinv


_tc_mid = pl.pallas_call(
    _tc_mid_body,
    grid=(NBLK,),
    in_specs=[
        pl.BlockSpec((RB, DH), lambda i: (i, 0)),
        pl.BlockSpec((RB, DH), lambda i: (i, 0)),
        pl.BlockSpec((RB, DH), lambda i: (i, 0)),
        pl.BlockSpec((RB, 16), lambda i: (i, 0)),
        pl.BlockSpec((1, DH), lambda i: (0, 0)),
        pl.BlockSpec((DH, DH), lambda i: (0, 0)),
    ],
    out_specs=pl.BlockSpec((RB, DH), lambda i: (i, 0)),
    out_shape=jax.ShapeDtypeStruct((N, DH), jnp.float32),
)


def _tc_fin_body(a0_ref, a1_ref, u_ref, dinv_ref, b_ref, batch_ref,
                 dw0_ref, db0_ref, dw1_ref, db1_ref, out_ref, pool_ref):
    i = pl.program_id(0)

    @pl.when(i == 0)
    def _():
        pool_ref[...] = jnp.zeros_like(pool_ref)

    dinv = dinv_ref[...][:, :1]
    h = (a0_ref[...] + a1_ref[...] + u_ref[...]) * dinv + b_ref[...]
    h = jnp.maximum(h, 0.0)
    row = batch_ref[0, 0, :]
    seg = (row[None, :] == lax.broadcasted_iota(jnp.int32, (G, RB), 0))
    seg = seg.astype(jnp.float32)
    pool_ref[...] += jnp.dot(seg, h, preferred_element_type=jnp.float32)

    @pl.when(i == NBLK - 1)
    def _():
        z = jnp.dot(pool_ref[...], dw0_ref[...],
                    preferred_element_type=jnp.float32) + db0_ref[...]
        z = jnp.maximum(z, 0.0)
        out_ref[...] = jnp.dot(z, dw1_ref[...],
                               preferred_element_type=jnp.float32) + db1_ref[...]


_tc_fin = pl.pallas_call(
    _tc_fin_body,
    grid=(NBLK,),
    in_specs=[
        pl.BlockSpec((RB, DH), lambda i: (i, 0)),
        pl.BlockSpec((RB, DH), lambda i: (i, 0)),
        pl.BlockSpec((RB, DH), lambda i: (i, 0)),
        pl.BlockSpec((RB, 16), lambda i: (i, 0)),
        pl.BlockSpec((1, DH), lambda i: (0, 0)),
        pl.BlockSpec((1, 1, RB), lambda i: (i, 0, 0)),
        pl.BlockSpec((DH, DH), lambda i: (0, 0)),
        pl.BlockSpec((1, DH), lambda i: (0, 0)),
        pl.BlockSpec((DH, DOUT), lambda i: (0, 0)),
        pl.BlockSpec((1, DOUT), lambda i: (0, 0)),
    ],
    out_specs=pl.BlockSpec((G, DOUT), lambda i: (0, 0)),
    out_shape=jax.ShapeDtypeStruct((G, DOUT), jnp.float32),
    scratch_shapes=[pltpu.VMEM((G, DH), jnp.float32)],
)


def kernel(x, edge_index, batch, enc_W, enc_b, conv_W, conv_b,
           dec_W0, dec_b0, dec_W1, dec_b1):
    src = edge_index[0].reshape(NW, NSUP, SUP, C)
    dst = edge_index[1].reshape(NW, NSUP, SUP, C)
    dst_deg = edge_index[1].reshape(NW, NCHUNK, C)
    zeros128 = jnp.zeros((NP, DH), jnp.float32)
    ones128 = jnp.ones((C, DH), jnp.float32)
    batch3 = batch.reshape(NBLK, 1, RB)

    _sc_deg, _sc_agg = _sc_kernels()
    deg = _sc_deg(dst_deg, ones128, zeros128)
    u, dinv = _tc_enc(x, enc_W[0], enc_b[0:1], enc_W[1], enc_b[1:2],
                      conv_W[0], deg[0], deg[1])

    for i in range(3):
        agg = _sc_agg(u, src, dst, zeros128)
        if i < 2:
            u = _tc_mid(agg[0], agg[1], u, dinv, conv_b[i:i + 1],
                        conv_W[i + 1])
        else:
            out = _tc_fin(agg[0], agg[1], u, dinv, conv_b[i:i + 1], batch3,
                          dec_W0, dec_b0[None], dec_W1, dec_b1[None])
    return out
